# baseline (speedup 1.0000x reference)
import math

import jax
import jax.numpy as jnp
from jax import lax
from jax.experimental import pallas as pl
from jax.experimental.pallas import tpu as pltpu

_P = 8
_CHUNK = 2048
_INV_SQRT_2PI_CUBED = 1.0 / (2.0 * math.pi) ** 1.5


def _make_body(n, npad):
    def _raster_body(par_ref, spt_ref, ns_ref, out_ref, off_ref,
                     e0_ref, e1_ref, e2_ref):
        f32 = jnp.float32
        i = pl.program_id(0)
        j = pl.program_id(1)

        @pl.when((i == 0) & (j == 0))
        def _build_tables():
            ns = ns_ref[0:1, 0:1]
            scol = spt_ref[:, :]
            ones = jnp.ones((3, _CHUNK), f32)
            s3 = scol * ones
            rs3 = 1.0 / s3
            kp = (spt_ref[0:1, 0:1] * spt_ref[1:2, 0:1] * spt_ref[2:3, 0:1]
                  * _INV_SQRT_2PI_CUBED)
            ii8 = lax.broadcasted_iota(jnp.int32, (_P, _CHUNK), 0).astype(f32)

            for c in range(npad // _CHUNK):
                lane0 = c * _CHUNK
                sig3 = par_ref[0:3, lane0:lane0 + _CHUNK]
                c3 = par_ref[3:6, lane0:lane0 + _CHUNK]
                chg = par_ref[6:7, lane0:lane0 + _CHUNK]
                lo3 = jnp.floor((c3 - ns * sig3) * rs3)
                offs = lo3.astype(jnp.int32)
                if lane0 + _CHUNK <= n:
                    off_ref[:, lane0:lane0 + _CHUNK] = offs
                elif lane0 < n:
                    off_ref[:, lane0:n] = offs[:, :n - lane0]
                inv3 = 1.0 / sig3
                m3 = s3 * inv3
                a3 = ((lo3 + 0.5) * s3 - c3) * inv3
                amp = (chg * kp * inv3[0:1, :] * inv3[1:2, :]
                       * inv3[2:3, :])
                z0 = jnp.broadcast_to(a3[0:1, :], (_P, _CHUNK)) \
                    + jnp.broadcast_to(m3[0:1, :], (_P, _CHUNK)) * ii8
                z1 = jnp.broadcast_to(a3[1:2, :], (_P, _CHUNK)) \
                    + jnp.broadcast_to(m3[1:2, :], (_P, _CHUNK)) * ii8
                z2 = jnp.broadcast_to(a3[2:3, :], (_P, _CHUNK)) \
                    + jnp.broadcast_to(m3[2:3, :], (_P, _CHUNK)) * ii8
                e0_ref[:, lane0:lane0 + _CHUNK] = (
                    jnp.broadcast_to(amp, (_P, _CHUNK))
                    * jnp.exp(-0.5 * (z0 * z0)))
                e1_ref[:, lane0:lane0 + _CHUNK] = jnp.exp(-0.5 * (z1 * z1))
                e2_ref[:, lane0:lane0 + _CHUNK] = jnp.exp(-0.5 * (z2 * z2))

        g01 = e0_ref[pl.ds(i, 1), :n] * e1_ref[pl.ds(j, 1), :n]
        out_ref[0, 0, :, :] = e2_ref[:, :n] * g01

    return _raster_body


def kernel(sigma, time, charge, tail, grid_spacing, nsigma):
    n = sigma.shape[0]
    npad = -(-n // _CHUNK) * _CHUNK
    params = jnp.stack(
        [sigma[:, 0], sigma[:, 1], sigma[:, 2],
         tail[:, 1], tail[:, 2], time, charge,
         jnp.zeros_like(time)], axis=0)
    params = jnp.pad(params, ((0, 0), (0, npad - n)), constant_values=1.0)
    rasters, offsets = pl.pallas_call(
        _make_body(n, npad),
        grid=(_P, _P),
        in_specs=[
            pl.BlockSpec((8, npad), lambda i, j: (0, 0)),
            pl.BlockSpec((3, 1), lambda i, j: (0, 0)),
            pl.BlockSpec((1, 1), lambda i, j: (0, 0)),
        ],
        out_specs=[
            pl.BlockSpec((1, 1, _P, n), lambda i, j: (i, j, 0, 0)),
            pl.BlockSpec((3, n), lambda i, j: (0, 0)),
        ],
        out_shape=[
            jax.ShapeDtypeStruct((_P, _P, _P, n), jnp.float32),
            jax.ShapeDtypeStruct((3, n), jnp.int32),
        ],
        scratch_shapes=[
            pltpu.VMEM((_P, npad), jnp.float32),
            pltpu.VMEM((_P, npad), jnp.float32),
            pltpu.VMEM((_P, npad), jnp.float32),
        ],
    )(params, grid_spacing.reshape(3, 1), jnp.reshape(nsigma, (1, 1)))
    return rasters.transpose(3, 0, 1, 2), offsets.T


# speedup vs baseline: 1.1119x; 1.1119x over previous
import math

import jax
import jax.numpy as jnp
from jax import lax
from jax.experimental import pallas as pl
from jax.experimental.pallas import tpu as pltpu

_P = 8
_B = 2048
_CH = 14336
_R = 12
_INV_SQRT_2PI_CUBED = 1.0 / (2.0 * math.pi) ** 1.5


def _make_body(n, npad):
    nfull = n // _CH
    tail = n - nfull * _CH

    def _raster_body(par_ref, spt_ref, ns_ref, out_ref, off_hbm, *rest):
        if tail:
            (e0_ref, e1_ref, e2_ref, off_ref, ring_ref, tring_ref,
             sems, tsems, osem) = rest
        else:
            (e0_ref, e1_ref, e2_ref, off_ref, ring_ref,
             sems, osem) = rest
        f32 = jnp.float32
        ns = ns_ref[0:1, 0:1]
        scol = spt_ref[:, :]
        ones = jnp.ones((3, _B), f32)
        s3 = scol * ones
        rs3 = 1.0 / s3
        kp = (spt_ref[0:1, 0:1] * spt_ref[1:2, 0:1] * spt_ref[2:3, 0:1]
              * _INV_SQRT_2PI_CUBED)
        ii8 = lax.broadcasted_iota(jnp.int32, (_P, _B), 0).astype(f32)

        def build(lane0):
            sig3 = par_ref[0:3, lane0:lane0 + _B]
            c3 = par_ref[3:6, lane0:lane0 + _B]
            chg = par_ref[6:7, lane0:lane0 + _B]
            lo3 = jnp.floor((c3 - ns * sig3) * rs3)
            offs = lo3.astype(jnp.int32)
            if lane0 + _B <= n:
                off_ref[:, lane0:lane0 + _B] = offs
            else:
                off_ref[:, lane0:n] = offs[:, :n - lane0]
            inv3 = 1.0 / sig3
            m3 = s3 * inv3
            a3 = ((lo3 + 0.5) * s3 - c3) * inv3
            amp = (chg * kp * inv3[0:1, :] * inv3[1:2, :]
                   * inv3[2:3, :])
            z0 = jnp.broadcast_to(a3[0:1, :], (_P, _B)) \
                + jnp.broadcast_to(m3[0:1, :], (_P, _B)) * ii8
            z1 = jnp.broadcast_to(a3[1:2, :], (_P, _B)) \
                + jnp.broadcast_to(m3[1:2, :], (_P, _B)) * ii8
            z2 = jnp.broadcast_to(a3[2:3, :], (_P, _B)) \
                + jnp.broadcast_to(m3[2:3, :], (_P, _B)) * ii8
            e0_ref[:, lane0:lane0 + _B] = (
                jnp.broadcast_to(amp, (_P, _B))
                * jnp.exp(-0.5 * (z0 * z0)))
            e1_ref[:, lane0:lane0 + _B] = jnp.exp(-0.5 * (z1 * z1))
            e2_ref[:, lane0:lane0 + _B] = jnp.exp(-0.5 * (z2 * z2))

        def copy(rref, sref, r, i, j, lane0, w):
            return pltpu.make_async_copy(
                rref.at[r],
                out_ref.at[i, j, :, pl.ds(lane0, w)],
                sref.at[r])

        inflight = {}
        nchunks = nfull + (1 if tail else 0)
        p = 0
        for c in range(nchunks):
            lane0 = c * _CH
            is_tail = tail and c == nfull
            w = tail if is_tail else _CH
            rref = tring_ref if is_tail else ring_ref
            sref = tsems if is_tail else sems
            for b in range(-(-w // _B)):
                build(lane0 + b * _B)
            if c == nchunks - 1:
                pltpu.make_async_copy(off_ref, off_hbm, osem).start()
            for i in range(_P):
                for j in range(_P):
                    r = p % _R
                    if (id(rref), r) in inflight:
                        copy(*inflight.pop((id(rref), r))).wait()
                    g = (e0_ref[i:i + 1, lane0:lane0 + w]
                         * e1_ref[j:j + 1, lane0:lane0 + w])
                    rref[r, :, :] = e2_ref[:, lane0:lane0 + w] * g
                    copy(rref, sref, r, i, j, lane0, w).start()
                    inflight[(id(rref), r)] = (rref, sref, r, i, j,
                                               lane0, w)
                    p += 1
        for key in list(inflight):
            copy(*inflight.pop(key)).wait()
        pltpu.make_async_copy(off_ref, off_hbm, osem).wait()

    return _raster_body


def kernel(sigma, time, charge, tail, grid_spacing, nsigma):
    n = sigma.shape[0]
    npad = -(-n // _B) * _B
    ntail = n - (n // _CH) * _CH
    params = jnp.stack(
        [sigma[:, 0], sigma[:, 1], sigma[:, 2],
         tail[:, 1], tail[:, 2], time, charge,
         jnp.zeros_like(time)], axis=0)
    params = jnp.pad(params, ((0, 0), (0, npad - n)), constant_values=1.0)
    scratch = [
        pltpu.VMEM((_P, npad), jnp.float32),
        pltpu.VMEM((_P, npad), jnp.float32),
        pltpu.VMEM((_P, npad), jnp.float32),
        pltpu.VMEM((3, n), jnp.int32),
        pltpu.VMEM((_R, _P, _CH), jnp.float32),
    ]
    if ntail:
        scratch.append(pltpu.VMEM((_R, _P, ntail), jnp.float32))
    scratch.append(pltpu.SemaphoreType.DMA((_R,)))
    if ntail:
        scratch.append(pltpu.SemaphoreType.DMA((_R,)))
    scratch.append(pltpu.SemaphoreType.DMA)
    rasters, offsets = pl.pallas_call(
        _make_body(n, npad),
        in_specs=[
            pl.BlockSpec((8, npad), lambda: (0, 0)),
            pl.BlockSpec((3, 1), lambda: (0, 0)),
            pl.BlockSpec((1, 1), lambda: (0, 0)),
        ],
        out_specs=[
            pl.BlockSpec(memory_space=pl.ANY),
            pl.BlockSpec(memory_space=pl.ANY),
        ],
        out_shape=[
            jax.ShapeDtypeStruct((_P, _P, _P, n), jnp.float32),
            jax.ShapeDtypeStruct((3, n), jnp.int32),
        ],
        scratch_shapes=scratch,
    )(params, grid_spacing.reshape(3, 1), jnp.reshape(nsigma, (1, 1)))
    return rasters.transpose(3, 0, 1, 2), offsets.T


# speedup vs baseline: 1.1254x; 1.0121x over previous
import math

import jax
import jax.numpy as jnp
from jax import lax
from jax.experimental import pallas as pl
from jax.experimental.pallas import tpu as pltpu

_P = 8
_B = 2048
_CH = 14336
_R = 20
_INV_SQRT_2PI_CUBED = 1.0 / (2.0 * math.pi) ** 1.5


def _make_body(n, npad):
    nfull = n // _CH
    tail = n - nfull * _CH

    widths = []
    lane = 0
    for w in (_B, 2 * _B, 4 * _B):
        if lane + w <= nfull * _CH:
            widths.append(w)
            lane += w
    while lane + _CH <= nfull * _CH:
        widths.append(_CH)
        lane += _CH
    if lane < nfull * _CH:
        widths.append(nfull * _CH - lane)
    if tail:
        widths.append(tail)

    def _raster_body(par_ref, spt_ref, ns_ref, out_ref, off_hbm, *rest):
        if tail:
            (e0_ref, e1_ref, e2_ref, off_ref, ring_ref, tring_ref,
             sems, tsems, osem) = rest
        else:
            (e0_ref, e1_ref, e2_ref, off_ref, ring_ref,
             sems, osem) = rest
        f32 = jnp.float32
        ns = ns_ref[0:1, 0:1]
        scol = spt_ref[:, :]
        ones = jnp.ones((3, _B), f32)
        s3 = scol * ones
        rs3 = 1.0 / s3
        kp = (spt_ref[0:1, 0:1] * spt_ref[1:2, 0:1] * spt_ref[2:3, 0:1]
              * _INV_SQRT_2PI_CUBED)
        ii8 = lax.broadcasted_iota(jnp.int32, (_P, _B), 0).astype(f32)

        def build(lane0):
            sig3 = par_ref[0:3, lane0:lane0 + _B]
            c3 = par_ref[3:6, lane0:lane0 + _B]
            chg = par_ref[6:7, lane0:lane0 + _B]
            lo3 = jnp.floor((c3 - ns * sig3) * rs3)
            offs = lo3.astype(jnp.int32)
            if lane0 + _B <= n:
                off_ref[:, lane0:lane0 + _B] = offs
            else:
                off_ref[:, lane0:n] = offs[:, :n - lane0]
            inv3 = 1.0 / sig3
            m3 = s3 * inv3
            a3 = ((lo3 + 0.5) * s3 - c3) * inv3
            amp = (chg * kp * inv3[0:1, :] * inv3[1:2, :]
                   * inv3[2:3, :])
            z0 = jnp.broadcast_to(a3[0:1, :], (_P, _B)) \
                + jnp.broadcast_to(m3[0:1, :], (_P, _B)) * ii8
            z1 = jnp.broadcast_to(a3[1:2, :], (_P, _B)) \
                + jnp.broadcast_to(m3[1:2, :], (_P, _B)) * ii8
            z2 = jnp.broadcast_to(a3[2:3, :], (_P, _B)) \
                + jnp.broadcast_to(m3[2:3, :], (_P, _B)) * ii8
            e0_ref[:, lane0:lane0 + _B] = (
                jnp.broadcast_to(amp, (_P, _B))
                * jnp.exp(-0.5 * (z0 * z0)))
            e1_ref[:, lane0:lane0 + _B] = jnp.exp(-0.5 * (z1 * z1))
            e2_ref[:, lane0:lane0 + _B] = jnp.exp(-0.5 * (z2 * z2))

        def copy(rref, sref, r, i, j, lane0, w):
            src = rref.at[r] if w == _CH or rref is not ring_ref \
                else rref.at[r, :, pl.ds(0, w)]
            return pltpu.make_async_copy(
                src,
                out_ref.at[i, j, :, pl.ds(lane0, w)],
                sref.at[r])

        inflight = {}
        p = 0
        lane0 = 0
        for c, w in enumerate(widths):
            is_tail = tail and c == len(widths) - 1
            rref = tring_ref if is_tail else ring_ref
            sref = tsems if is_tail else sems
            for b in range(-(-w // _B)):
                build(lane0 + b * _B)
            if c == len(widths) - 1:
                pltpu.make_async_copy(off_ref, off_hbm, osem).start()
            for i in range(_P):
                for j in range(_P):
                    r = p % _R
                    if (id(rref), r) in inflight:
                        copy(*inflight.pop((id(rref), r))).wait()
                    g = (e0_ref[i:i + 1, lane0:lane0 + w]
                         * e1_ref[j:j + 1, lane0:lane0 + w])
                    rref[r, :, pl.ds(0, w)] = \
                        e2_ref[:, lane0:lane0 + w] * g
                    copy(rref, sref, r, i, j, lane0, w).start()
                    inflight[(id(rref), r)] = (rref, sref, r, i, j,
                                               lane0, w)
                    p += 1
            lane0 += w
        for key in list(inflight):
            copy(*inflight.pop(key)).wait()
        pltpu.make_async_copy(off_ref, off_hbm, osem).wait()

    return _raster_body


def kernel(sigma, time, charge, tail, grid_spacing, nsigma):
    n = sigma.shape[0]
    npad = -(-n // _B) * _B
    ntail = n - (n // _CH) * _CH
    params = jnp.stack(
        [sigma[:, 0], sigma[:, 1], sigma[:, 2],
         tail[:, 1], tail[:, 2], time, charge,
         jnp.zeros_like(time)], axis=0)
    params = jnp.pad(params, ((0, 0), (0, npad - n)), constant_values=1.0)
    scratch = [
        pltpu.VMEM((_P, npad), jnp.float32),
        pltpu.VMEM((_P, npad), jnp.float32),
        pltpu.VMEM((_P, npad), jnp.float32),
        pltpu.VMEM((3, n), jnp.int32),
        pltpu.VMEM((_R, _P, _CH), jnp.float32),
    ]
    if ntail:
        scratch.append(pltpu.VMEM((_R, _P, ntail), jnp.float32))
    scratch.append(pltpu.SemaphoreType.DMA((_R,)))
    if ntail:
        scratch.append(pltpu.SemaphoreType.DMA((_R,)))
    scratch.append(pltpu.SemaphoreType.DMA)
    rasters, offsets = pl.pallas_call(
        _make_body(n, npad),
        in_specs=[
            pl.BlockSpec((8, npad), lambda: (0, 0)),
            pl.BlockSpec((3, 1), lambda: (0, 0)),
            pl.BlockSpec((1, 1), lambda: (0, 0)),
        ],
        out_specs=[
            pl.BlockSpec(memory_space=pl.ANY),
            pl.BlockSpec(memory_space=pl.ANY),
        ],
        out_shape=[
            jax.ShapeDtypeStruct((_P, _P, _P, n), jnp.float32),
            jax.ShapeDtypeStruct((3, n), jnp.int32),
        ],
        scratch_shapes=scratch,
    )(params, grid_spacing.reshape(3, 1), jnp.reshape(nsigma, (1, 1)))
    return rasters.transpose(3, 0, 1, 2), offsets.T


# speedup vs baseline: 1.5218x; 1.3523x over previous
import math

import jax
import jax.numpy as jnp
from jax import lax
from jax.experimental import pallas as pl
from jax.experimental.pallas import tpu as pltpu

_P = 8
_B = 2048
_CH = 14336
_R = 16
_RT = 8
_INV_SQRT_2PI_CUBED = 1.0 / (2.0 * math.pi) ** 1.5


def _make_body(n):
    nfull = n // _CH
    tail = n - nfull * _CH

    widths = []
    lane = 0
    for w in (_B, 2 * _B, 4 * _B):
        if lane + w <= nfull * _CH:
            widths.append(w)
            lane += w
    while lane + _CH <= nfull * _CH:
        widths.append(_CH)
        lane += _CH
    if lane < nfull * _CH:
        widths.append(nfull * _CH - lane)
    if tail:
        widths.append(tail)

    def _raster_body(sig_ref, tl_ref, tm_ref, chg_ref, spt_ref, ns_ref,
                     out_ref, off_hbm, *rest):
        if tail:
            (e0_ref, e1_ref, e2_ref, off_ref, ring_ref, tring_ref,
             sems, tsems, osem) = rest
        else:
            (e0_ref, e1_ref, e2_ref, off_ref, ring_ref,
             sems, osem) = rest
        ns = ns_ref[0]
        s0, s1, s2 = spt_ref[0], spt_ref[1], spt_ref[2]
        kp = s0 * s1 * s2 * _INV_SQRT_2PI_CUBED

        def build(lane0, bw):
            ii8 = lax.broadcasted_iota(
                jnp.int32, (_P, bw), 0).astype(jnp.float32)
            sl = pl.ds(lane0, bw)
            sg = [sig_ref[a:a + 1, sl] for a in range(3)]
            ctr = [tl_ref[1:2, sl], tl_ref[2:3, sl],
                   jnp.reshape(tm_ref[sl], (1, bw))]
            chg = jnp.reshape(chg_ref[sl], (1, bw))
            inv = [1.0 / g for g in sg]
            amp = chg * (kp * inv[0] * inv[1] * inv[2])
            es = [e0_ref, e1_ref, e2_ref]
            scale = [amp, None, None]
            for a, s_a in enumerate((s0, s1, s2)):
                lo = jnp.floor((ctr[a] - ns * sg[a]) * (1.0 / s_a))
                off_ref[a:a + 1, sl] = lo.astype(jnp.int32)
                m = s_a * inv[a]
                a0 = ((lo + 0.5) * s_a - ctr[a]) * inv[a]
                z = (jnp.broadcast_to(a0, (_P, bw))
                     + jnp.broadcast_to(m, (_P, bw)) * ii8)
                e = jnp.exp(-0.5 * (z * z))
                if scale[a] is not None:
                    e = e * jnp.broadcast_to(scale[a], (_P, bw))
                es[a][:, sl] = e

        def copy(rref, sref, r, i, j, lane0, w):
            src = rref.at[r] if w == _CH or rref is not ring_ref \
                else rref.at[r, :, pl.ds(0, w)]
            return pltpu.make_async_copy(
                src,
                out_ref.at[i, j, :, pl.ds(lane0, w)],
                sref.at[r])

        inflight = {}
        p = 0
        lane0 = 0
        for c, w in enumerate(widths):
            is_tail = tail and c == len(widths) - 1
            rref = tring_ref if is_tail else ring_ref
            sref = tsems if is_tail else sems
            for b0 in range(0, w, _B):
                build(lane0 + b0, min(_B, w - b0))
            if c == len(widths) - 1:
                pltpu.make_async_copy(off_ref, off_hbm, osem).start()
            depth = _RT if is_tail else _R
            for i in range(_P):
                for j in range(_P):
                    r = p % depth
                    if (id(rref), r) in inflight:
                        copy(*inflight.pop((id(rref), r))).wait()
                    g = (e0_ref[i:i + 1, lane0:lane0 + w]
                         * e1_ref[j:j + 1, lane0:lane0 + w])
                    rref[r, :, pl.ds(0, w)] = \
                        e2_ref[:, lane0:lane0 + w] * g
                    copy(rref, sref, r, i, j, lane0, w).start()
                    inflight[(id(rref), r)] = (rref, sref, r, i, j,
                                               lane0, w)
                    p += 1
            lane0 += w
        for key in list(inflight):
            copy(*inflight.pop(key)).wait()
        pltpu.make_async_copy(off_ref, off_hbm, osem).wait()

    return _raster_body


def kernel(sigma, time, charge, tail, grid_spacing, nsigma):
    n = sigma.shape[0]
    ntail = n - (n // _CH) * _CH
    scratch = [
        pltpu.VMEM((_P, n), jnp.float32),
        pltpu.VMEM((_P, n), jnp.float32),
        pltpu.VMEM((_P, n), jnp.float32),
        pltpu.VMEM((3, n), jnp.int32),
        pltpu.VMEM((_R, _P, _CH), jnp.float32),
    ]
    if ntail:
        scratch.append(pltpu.VMEM((_RT, _P, ntail), jnp.float32))
    scratch.append(pltpu.SemaphoreType.DMA((_R,)))
    if ntail:
        scratch.append(pltpu.SemaphoreType.DMA((_RT,)))
    scratch.append(pltpu.SemaphoreType.DMA)
    rasters, offsets = pl.pallas_call(
        _make_body(n),
        in_specs=[
            pl.BlockSpec((3, n), lambda: (0, 0)),
            pl.BlockSpec((3, n), lambda: (0, 0)),
            pl.BlockSpec((n,), lambda: (0,)),
            pl.BlockSpec((n,), lambda: (0,)),
            pl.BlockSpec(memory_space=pltpu.SMEM),
            pl.BlockSpec(memory_space=pltpu.SMEM),
        ],
        out_specs=[
            pl.BlockSpec(memory_space=pl.ANY),
            pl.BlockSpec(memory_space=pl.ANY),
        ],
        out_shape=[
            jax.ShapeDtypeStruct((_P, _P, _P, n), jnp.float32),
            jax.ShapeDtypeStruct((3, n), jnp.int32),
        ],
        scratch_shapes=scratch,
    )(sigma.T, tail.T, time, charge, grid_spacing,
      jnp.reshape(nsigma, (1,)))
    return rasters.transpose(3, 0, 1, 2), offsets.T


# speedup vs baseline: 1.5563x; 1.0227x over previous
import math

import jax
import jax.numpy as jnp
from jax import lax
from jax.experimental import pallas as pl
from jax.experimental.pallas import tpu as pltpu

_P = 8
_B = 2048
_CH = 14336
_R = 24
_RT = 8
_INV_SQRT_2PI_CUBED = 1.0 / (2.0 * math.pi) ** 1.5


def _make_body(n):
    nfull = n // _CH
    tail = n - nfull * _CH

    widths = []
    lane = 0
    for w in (_B, 2 * _B, 4 * _B):
        if lane + w <= nfull * _CH:
            widths.append(w)
            lane += w
    while lane + _CH <= nfull * _CH:
        widths.append(_CH)
        lane += _CH
    if lane < nfull * _CH:
        widths.append(nfull * _CH - lane)
    if tail:
        widths.append(tail)

    def _raster_body(sig_ref, tl_ref, tm_ref, chg_ref, spt_ref, ns_ref,
                     out_ref, off_hbm, *rest):
        if tail:
            (e0_ref, e1_ref, e2_ref, off_ref, ring_ref, tring_ref,
             sems, tsems, osem) = rest
        else:
            (e0_ref, e1_ref, e2_ref, off_ref, ring_ref,
             sems, osem) = rest
        ns = ns_ref[0]
        s0, s1, s2 = spt_ref[0], spt_ref[1], spt_ref[2]
        kp = s0 * s1 * s2 * _INV_SQRT_2PI_CUBED

        def build(lane0, bw):
            ii8 = lax.broadcasted_iota(
                jnp.int32, (_P, bw), 0).astype(jnp.float32)
            sl = pl.ds(lane0, bw)
            sg = [sig_ref[a:a + 1, sl] for a in range(3)]
            ctr = [tl_ref[1:2, sl], tl_ref[2:3, sl],
                   jnp.reshape(tm_ref[sl], (1, bw))]
            chg = jnp.reshape(chg_ref[sl], (1, bw))
            inv = [1.0 / g for g in sg]
            amp = chg * (kp * inv[0] * inv[1] * inv[2])
            es = [e0_ref, e1_ref, e2_ref]
            scale = [amp, None, None]
            for a, s_a in enumerate((s0, s1, s2)):
                lo = jnp.floor((ctr[a] - ns * sg[a]) * (1.0 / s_a))
                off_ref[a:a + 1, sl] = lo.astype(jnp.int32)
                m = s_a * inv[a]
                a0 = ((lo + 0.5) * s_a - ctr[a]) * inv[a]
                z = (jnp.broadcast_to(a0, (_P, bw))
                     + jnp.broadcast_to(m, (_P, bw)) * ii8)
                e = jnp.exp(-0.5 * (z * z))
                if scale[a] is not None:
                    e = e * jnp.broadcast_to(scale[a], (_P, bw))
                es[a][:, sl] = e

        def copy(rref, sref, r, i, j, lane0, w):
            src = rref.at[r] if w == _CH or rref is not ring_ref \
                else rref.at[r, :, pl.ds(0, w)]
            return pltpu.make_async_copy(
                src,
                out_ref.at[i, j, :, pl.ds(lane0, w)],
                sref.at[r])

        inflight = {}
        p = 0
        lane0 = 0
        for c, w in enumerate(widths):
            is_tail = tail and c == len(widths) - 1
            rref = tring_ref if is_tail else ring_ref
            sref = tsems if is_tail else sems
            for b0 in range(0, w, _B):
                build(lane0 + b0, min(_B, w - b0))
            if c == len(widths) - 1:
                pltpu.make_async_copy(off_ref, off_hbm, osem).start()
            depth = _RT if is_tail else _R
            for i in range(_P):
                for j in range(_P):
                    r = p % depth
                    if (id(rref), r) in inflight:
                        copy(*inflight.pop((id(rref), r))).wait()
                    g = (e0_ref[i:i + 1, lane0:lane0 + w]
                         * e1_ref[j:j + 1, lane0:lane0 + w])
                    rref[r, :, pl.ds(0, w)] = \
                        e2_ref[:, lane0:lane0 + w] * g
                    copy(rref, sref, r, i, j, lane0, w).start()
                    inflight[(id(rref), r)] = (rref, sref, r, i, j,
                                               lane0, w)
                    p += 1
            lane0 += w
        for key in list(inflight):
            copy(*inflight.pop(key)).wait()
        pltpu.make_async_copy(off_ref, off_hbm, osem).wait()

    return _raster_body


def kernel(sigma, time, charge, tail, grid_spacing, nsigma):
    n = sigma.shape[0]
    ntail = n - (n // _CH) * _CH
    scratch = [
        pltpu.VMEM((_P, n), jnp.float32),
        pltpu.VMEM((_P, n), jnp.float32),
        pltpu.VMEM((_P, n), jnp.float32),
        pltpu.VMEM((3, n), jnp.int32),
        pltpu.VMEM((_R, _P, _CH), jnp.float32),
    ]
    if ntail:
        scratch.append(pltpu.VMEM((_RT, _P, ntail), jnp.float32))
    scratch.append(pltpu.SemaphoreType.DMA((_R,)))
    if ntail:
        scratch.append(pltpu.SemaphoreType.DMA((_RT,)))
    scratch.append(pltpu.SemaphoreType.DMA)
    rasters, offsets = pl.pallas_call(
        _make_body(n),
        in_specs=[
            pl.BlockSpec((3, n), lambda: (0, 0)),
            pl.BlockSpec((3, n), lambda: (0, 0)),
            pl.BlockSpec((n,), lambda: (0,)),
            pl.BlockSpec((n,), lambda: (0,)),
            pl.BlockSpec(memory_space=pltpu.SMEM),
            pl.BlockSpec(memory_space=pltpu.SMEM),
        ],
        out_specs=[
            pl.BlockSpec(memory_space=pl.ANY),
            pl.BlockSpec(memory_space=pl.ANY),
        ],
        out_shape=[
            jax.ShapeDtypeStruct((_P, _P, _P, n), jnp.float32),
            jax.ShapeDtypeStruct((3, n), jnp.int32),
        ],
        scratch_shapes=scratch,
    )(sigma.T, tail.T, time, charge, grid_spacing,
      jnp.reshape(nsigma, (1,)))
    return rasters.transpose(3, 0, 1, 2), offsets.T
